# baseline (speedup 1.0000x reference)
import functools

import jax
import jax.numpy as jnp
from jax import lax
from jax.experimental import pallas as pl
from jax.experimental.pallas import tpu as pltpu
from jax.experimental.pallas import tpu_sc as plsc

N_NODES = 10000
D = 128
DH = D // 2
NC = 2
NS = 16
CHUNK = 64
GRP = 16
NBUF = 8
N_PAD = 10112
BAND = N_PAD // NS
DEGW = 16


def _sc_aggregate(hsplit, src, dst, groups_per_tile):
    mesh = plsc.VectorSubcoreMesh(core_axis_name="c", subcore_axis_name="s")

    @functools.partial(
        pl.kernel,
        out_type=[
            jax.ShapeDtypeStruct((NC, N_PAD, DH), jnp.float32),
            jax.ShapeDtypeStruct((NC, N_PAD, DEGW), jnp.float32),
        ],
        mesh=mesh,
        compiler_params=pltpu.CompilerParams(use_tc_tiling_on_sc=False),
        scratch_types=[
            pltpu.VMEM((2, GRP, CHUNK), jnp.int32),
            pltpu.VMEM((2, GRP, CHUNK), jnp.int32),
            pltpu.VMEM((NBUF, CHUNK, DH), jnp.float32),
            pltpu.VMEM((CHUNK, DEGW), jnp.float32),
            pltpu.VMEM_SHARED((N_PAD, DH), jnp.float32),
            pltpu.VMEM_SHARED((N_PAD, DH), jnp.float32),
            pltpu.VMEM_SHARED((N_PAD, DEGW), jnp.float32),
            pltpu.SemaphoreType.DMA,
            [pltpu.SemaphoreType.DMA] * NBUF,
            [pltpu.SemaphoreType.DMA] * NBUF,
            pltpu.SemaphoreType.DMA,
        ],
    )
    def agg_kernel(h_hbm, src_hbm, dst_hbm, zagg_hbm, zdeg_hbm,
                   agg_out, deg_out,
                   src_v, dst_v, rows_v, ones_v,
                   h_sh, agg_sh, deg_sh, isem, gsems, ssems, dsem):
        cid = lax.axis_index("c")
        sid = lax.axis_index("s")
        deg_half = groups_per_tile // 2

        ones16 = jnp.ones((16,), jnp.float32)

        def fill_ones(i, _):
            ones_v[i] = ones16
            return 0

        lax.fori_loop(0, CHUNK, fill_ones, 0)

        pltpu.async_copy(src_hbm.at[cid, sid, pl.ds(0, GRP)], src_v.at[0],
                         isem)
        pltpu.async_copy(dst_hbm.at[sid, pl.ds(0, GRP)], dst_v.at[0], isem)

        base = sid * BAND
        hd = pltpu.async_copy(h_hbm.at[pl.ds(cid * N_PAD + base, BAND)],
                              h_sh.at[pl.ds(base, BAND)], gsems[0])
        hd.wait()
        za = pltpu.async_copy(zagg_hbm, agg_sh.at[pl.ds(base, BAND)], gsems[1])
        zd = pltpu.async_copy(zdeg_hbm, deg_sh.at[pl.ds(base, BAND)], gsems[2])
        za.wait()
        zd.wait()
        plsc.subcore_barrier()

        def group(g, _):
            cur = lax.rem(g, 2)
            sv = src_v.at[cur]
            dv = dst_v.at[cur]
            pltpu.make_async_copy(
                src_hbm.at[cid, sid, pl.ds(g * GRP, GRP)], sv, isem).wait()
            pltpu.make_async_copy(
                dst_hbm.at[sid, pl.ds(g * GRP, GRP)], dv, isem).wait()

            @pl.when(g + 1 < groups_per_tile)
            def _prefetch():
                nxt = 1 - cur
                pltpu.async_copy(
                    src_hbm.at[cid, sid, pl.ds((g + 1) * GRP, GRP)],
                    src_v.at[nxt], isem)
                pltpu.async_copy(
                    dst_hbm.at[sid, pl.ds((g + 1) * GRP, GRP)],
                    dst_v.at[nxt], isem)

            do_deg = lax.select(cid == 0, g < deg_half, g >= deg_half)

            @pl.when(do_deg)
            def _deg_scatter():
                for j in range(GRP):
                    pltpu.async_copy(ones_v, deg_sh.at[dv.at[j]], dsem,
                                     add=True)

            def gather(j, b):
                if j % 4 == 3:
                    return pltpu.async_copy(
                        h_hbm.at[sv.at[j]], rows_v.at[b], gsems[b])
                return pltpu.async_copy(
                    h_sh.at[sv.at[j]], rows_v.at[b], gsems[b])

            gd = {}
            sd = {}
            for j in range(NBUF - 1):
                gd[j] = gather(j, j)
            for j in range(GRP):
                b = j % NBUF
                jn = j + NBUF - 1
                if jn < GRP:
                    bn = jn % NBUF
                    if j >= 1:
                        sd[j - 1].wait()
                    gd[jn] = gather(jn, bn)
                gd[j].wait()
                sd[j] = pltpu.async_copy(
                    rows_v.at[b], agg_sh.at[dv.at[j]], ssems[b], add=True)
            for j in range(GRP - NBUF, GRP):
                sd[j].wait()

            @pl.when(do_deg)
            def _deg_drain():
                for j in range(GRP):
                    pltpu.make_async_copy(
                        ones_v, deg_sh.at[dv.at[j]], dsem).wait()

            return 0

        lax.fori_loop(0, groups_per_tile, group, 0)
        plsc.subcore_barrier()

        pltpu.sync_copy(agg_sh.at[pl.ds(base, BAND)],
                        agg_out.at[cid, pl.ds(base, BAND)])
        pltpu.sync_copy(deg_sh.at[pl.ds(base, BAND)],
                        deg_out.at[cid, pl.ds(base, BAND)])

    zagg = jnp.zeros((BAND, DH), jnp.float32)
    zdeg = jnp.zeros((BAND, DEGW), jnp.float32)
    return agg_kernel(hsplit, src, dst, zagg, zdeg)


def _tc_body(h_ref, a0_ref, a1_ref, d0_ref, d1_ref, w_ref, b_ref, o_ref):
    deg = d0_ref[0, :, 0:1] + d1_ref[0, :, 0:1]
    scale = 1.0 / jnp.maximum(deg, 1.0)
    gate = deg > 0.0
    hn0 = jnp.where(gate, a0_ref[0] * scale, h_ref[:, 0:DH])
    hn1 = jnp.where(gate, a1_ref[0] * scale, h_ref[:, DH:D])
    acc = (jnp.dot(hn0, w_ref[0:DH, :], preferred_element_type=jnp.float32)
           + jnp.dot(hn1, w_ref[DH:D, :], preferred_element_type=jnp.float32))
    o_ref[...] = jnp.maximum(acc + b_ref[...], 0.0)


def _tc_update(h, agg, deg, W, b):
    R = 2000
    grid = (N_NODES // R,)
    return pl.pallas_call(
        _tc_body,
        grid=grid,
        in_specs=[
            pl.BlockSpec((R, D), lambda i: (i, 0)),
            pl.BlockSpec((1, R, DH), lambda i: (0, i, 0)),
            pl.BlockSpec((1, R, DH), lambda i: (1, i, 0)),
            pl.BlockSpec((1, R, DEGW), lambda i: (0, i, 0)),
            pl.BlockSpec((1, R, DEGW), lambda i: (1, i, 0)),
            pl.BlockSpec((D, D), lambda i: (0, 0)),
            pl.BlockSpec((1, D), lambda i: (0, 0)),
        ],
        out_specs=pl.BlockSpec((R, D), lambda i: (i, 0)),
        out_shape=jax.ShapeDtypeStruct((N_NODES, D), jnp.float32),
    )(h, agg, agg, deg, deg, W, b)


def kernel(h, edge_index, W, b):
    src = edge_index[0].astype(jnp.int32)
    dst = edge_index[1].astype(jnp.int32)
    E = src.shape[0]
    lane = NS * GRP * CHUNK
    groups_per_tile = -(-E // lane)
    e_pad = lane * groups_per_tile
    if e_pad != E:
        src = jnp.concatenate(
            [src, jnp.zeros((e_pad - E,), jnp.int32)])
        dst = jnp.concatenate(
            [dst, jnp.full((e_pad - E,), N_NODES, jnp.int32)])
    n_chunks = groups_per_tile * GRP
    src = src.reshape(NS, n_chunks, CHUNK)
    dst = dst.reshape(NS, n_chunks, CHUNK)
    hbm_chunk = (jnp.arange(n_chunks, dtype=jnp.int32) % 4 == 3)
    offs = jnp.where(hbm_chunk, jnp.int32(N_PAD), jnp.int32(0))
    src2 = jnp.stack([src, src + offs[None, :, None]])

    hp = jnp.concatenate(
        [h, jnp.zeros((N_PAD - N_NODES, D), jnp.float32)])
    hsplit = hp.reshape(N_PAD, NC, DH).transpose(1, 0, 2)
    hflat = hsplit.reshape(NC * N_PAD, DH)

    agg_p, deg_p = _sc_aggregate(hflat, src2, dst, groups_per_tile)

    return _tc_update(h, agg_p, deg_p, W, b.reshape(1, D))


# speedup vs baseline: 1.2035x; 1.2035x over previous
import functools

import jax
import jax.numpy as jnp
from jax import lax
from jax.experimental import pallas as pl
from jax.experimental.pallas import tpu as pltpu
from jax.experimental.pallas import tpu_sc as plsc

N_NODES = 10000
D = 128
DH = D // 2
NC = 2
NS = 16
CHUNK = 64
GRP = 16
NBUF = 8
N_PAD = 10112
BAND = N_PAD // NS
DEGW = 16


def _sc_aggregate(hsplit, src, dst, groups_per_tile):
    mesh = plsc.VectorSubcoreMesh(core_axis_name="c", subcore_axis_name="s")

    @functools.partial(
        pl.kernel,
        out_type=[
            jax.ShapeDtypeStruct((NC, N_PAD, DH), jnp.float32),
            jax.ShapeDtypeStruct((NC, N_PAD, DEGW), jnp.float32),
        ],
        mesh=mesh,
        compiler_params=pltpu.CompilerParams(use_tc_tiling_on_sc=False),
        scratch_types=[
            pltpu.VMEM((2, GRP, CHUNK), jnp.int32),
            pltpu.VMEM((2, GRP, CHUNK), jnp.int32),
            pltpu.VMEM((NBUF, CHUNK, DH), jnp.float32),
            pltpu.VMEM((CHUNK, DEGW), jnp.float32),
            pltpu.VMEM_SHARED((N_PAD, DH), jnp.float32),
            pltpu.VMEM_SHARED((N_PAD, DH), jnp.float32),
            pltpu.VMEM_SHARED((N_PAD, DEGW), jnp.float32),
            pltpu.SemaphoreType.DMA,
            [pltpu.SemaphoreType.DMA] * NBUF,
            [pltpu.SemaphoreType.DMA] * NBUF,
            pltpu.SemaphoreType.DMA,
        ],
    )
    def agg_kernel(h_hbm, src_hbm, dst_hbm, zagg_hbm, zdeg_hbm,
                   agg_out, deg_out,
                   src_v, dst_v, rows_v, ones_v,
                   h_sh, agg_sh, deg_sh, isem, gsems, ssems, dsem):
        cid = lax.axis_index("c")
        sid = lax.axis_index("s")
        deg_half = groups_per_tile // 2

        ones16 = jnp.ones((16,), jnp.float32)

        def fill_ones(i, _):
            ones_v[i] = ones16
            return 0

        lax.fori_loop(0, CHUNK, fill_ones, 0)

        pltpu.async_copy(src_hbm.at[sid, pl.ds(0, GRP)], src_v.at[0], isem)
        pltpu.async_copy(dst_hbm.at[sid, pl.ds(0, GRP)], dst_v.at[0], isem)

        base = sid * BAND
        col = cid * DH

        @pl.when(base + BAND <= N_NODES)
        def _stage_full():
            pltpu.async_copy(h_hbm.at[pl.ds(base, BAND), pl.ds(col, DH)],
                             h_sh.at[pl.ds(base, BAND)], gsems[0]).wait()

        @pl.when(base + BAND > N_NODES)
        def _stage_tail():
            tail = N_NODES - (NS - 1) * BAND
            pltpu.async_copy(
                h_hbm.at[pl.ds((NS - 1) * BAND, tail), pl.ds(col, DH)],
                h_sh.at[pl.ds((NS - 1) * BAND, tail)], gsems[0]).wait()
        za = pltpu.async_copy(zagg_hbm, agg_sh.at[pl.ds(base, BAND)], gsems[1])
        zd = pltpu.async_copy(zdeg_hbm, deg_sh.at[pl.ds(base, BAND)], gsems[2])
        za.wait()
        zd.wait()
        plsc.subcore_barrier()

        def group(g, _):
            cur = lax.rem(g, 2)
            sv = src_v.at[cur]
            dv = dst_v.at[cur]
            pltpu.make_async_copy(
                src_hbm.at[sid, pl.ds(g * GRP, GRP)], sv, isem).wait()
            pltpu.make_async_copy(
                dst_hbm.at[sid, pl.ds(g * GRP, GRP)], dv, isem).wait()

            @pl.when(g + 1 < groups_per_tile)
            def _prefetch():
                nxt = 1 - cur
                pltpu.async_copy(
                    src_hbm.at[sid, pl.ds((g + 1) * GRP, GRP)],
                    src_v.at[nxt], isem)
                pltpu.async_copy(
                    dst_hbm.at[sid, pl.ds((g + 1) * GRP, GRP)],
                    dst_v.at[nxt], isem)

            do_deg = lax.select(cid == 0, g < deg_half, g >= deg_half)

            @pl.when(do_deg)
            def _deg_scatter():
                for j in range(GRP):
                    pltpu.async_copy(ones_v, deg_sh.at[dv.at[j]], dsem,
                                     add=True)

            def gather(j, b):
                return pltpu.async_copy(
                    h_sh.at[sv.at[j]], rows_v.at[b], gsems[b])

            gd = {}
            sd = {}
            for j in range(NBUF - 1):
                gd[j] = gather(j, j)
            for j in range(GRP):
                b = j % NBUF
                jn = j + NBUF - 1
                if jn < GRP:
                    bn = jn % NBUF
                    if j >= 1:
                        sd[j - 1].wait()
                    gd[jn] = gather(jn, bn)
                gd[j].wait()
                sd[j] = pltpu.async_copy(
                    rows_v.at[b], agg_sh.at[dv.at[j]], ssems[b], add=True)
            for j in range(GRP - NBUF, GRP):
                sd[j].wait()

            @pl.when(do_deg)
            def _deg_drain():
                for j in range(GRP):
                    pltpu.make_async_copy(
                        ones_v, deg_sh.at[dv.at[j]], dsem).wait()

            return 0

        lax.fori_loop(0, groups_per_tile, group, 0)
        plsc.subcore_barrier()

        pltpu.sync_copy(agg_sh.at[pl.ds(base, BAND)],
                        agg_out.at[cid, pl.ds(base, BAND)])
        pltpu.sync_copy(deg_sh.at[pl.ds(base, BAND)],
                        deg_out.at[cid, pl.ds(base, BAND)])

    zagg = jnp.zeros((BAND, DH), jnp.float32)
    zdeg = jnp.zeros((BAND, DEGW), jnp.float32)
    return agg_kernel(hsplit, src, dst, zagg, zdeg)


def _tc_body(h_ref, a0_ref, a1_ref, d0_ref, d1_ref, w_ref, b_ref, o_ref):
    deg = d0_ref[0, :, 0:1] + d1_ref[0, :, 0:1]
    scale = 1.0 / jnp.maximum(deg, 1.0)
    gate = deg > 0.0
    hn0 = jnp.where(gate, a0_ref[0] * scale, h_ref[:, 0:DH])
    hn1 = jnp.where(gate, a1_ref[0] * scale, h_ref[:, DH:D])
    acc = (jnp.dot(hn0, w_ref[0:DH, :], preferred_element_type=jnp.float32)
           + jnp.dot(hn1, w_ref[DH:D, :], preferred_element_type=jnp.float32))
    o_ref[...] = jnp.maximum(acc + b_ref[...], 0.0)


def _tc_update(h, agg, deg, W, b):
    R = 2000
    grid = (N_NODES // R,)
    return pl.pallas_call(
        _tc_body,
        grid=grid,
        in_specs=[
            pl.BlockSpec((R, D), lambda i: (i, 0)),
            pl.BlockSpec((1, R, DH), lambda i: (0, i, 0)),
            pl.BlockSpec((1, R, DH), lambda i: (1, i, 0)),
            pl.BlockSpec((1, R, DEGW), lambda i: (0, i, 0)),
            pl.BlockSpec((1, R, DEGW), lambda i: (1, i, 0)),
            pl.BlockSpec((D, D), lambda i: (0, 0)),
            pl.BlockSpec((1, D), lambda i: (0, 0)),
        ],
        out_specs=pl.BlockSpec((R, D), lambda i: (i, 0)),
        out_shape=jax.ShapeDtypeStruct((N_NODES, D), jnp.float32),
    )(h, agg, agg, deg, deg, W, b)


def kernel(h, edge_index, W, b):
    src = edge_index[0].astype(jnp.int32)
    dst = edge_index[1].astype(jnp.int32)
    E = src.shape[0]
    lane = NS * GRP * CHUNK
    groups_per_tile = -(-E // lane)
    e_pad = lane * groups_per_tile
    if e_pad != E:
        src = jnp.concatenate(
            [src, jnp.zeros((e_pad - E,), jnp.int32)])
        dst = jnp.concatenate(
            [dst, jnp.full((e_pad - E,), N_NODES, jnp.int32)])
    src = src.reshape(NS, groups_per_tile * GRP, CHUNK)
    dst = dst.reshape(NS, groups_per_tile * GRP, CHUNK)

    agg_p, deg_p = _sc_aggregate(h, src, dst, groups_per_tile)

    return _tc_update(h, agg_p, deg_p, W, b.reshape(1, D))


# speedup vs baseline: 1.2058x; 1.0019x over previous
import functools

import jax
import jax.numpy as jnp
from jax import lax
from jax.experimental import pallas as pl
from jax.experimental.pallas import tpu as pltpu
from jax.experimental.pallas import tpu_sc as plsc

N_NODES = 10000
D = 128
DH = D // 2
NC = 2
NS = 16
CHUNK = 64
GRP = 16
NBUF = 8
N_PAD = 10112
BAND = N_PAD // NS
DEGW = 16


def _sc_aggregate(hsplit, src, dst, groups_per_tile):
    mesh = plsc.VectorSubcoreMesh(core_axis_name="c", subcore_axis_name="s")

    @functools.partial(
        pl.kernel,
        out_type=[
            jax.ShapeDtypeStruct((NC, N_PAD, DH), jnp.float32),
            jax.ShapeDtypeStruct((NC, N_PAD, DEGW), jnp.float32),
        ],
        mesh=mesh,
        compiler_params=pltpu.CompilerParams(use_tc_tiling_on_sc=False),
        scratch_types=[
            pltpu.VMEM((2, GRP, CHUNK), jnp.int32),
            pltpu.VMEM((2, GRP, CHUNK), jnp.int32),
            pltpu.VMEM((NBUF, CHUNK, DH), jnp.float32),
            pltpu.VMEM((CHUNK, DEGW), jnp.float32),
            pltpu.VMEM_SHARED((N_PAD, DH), jnp.float32),
            pltpu.VMEM_SHARED((N_PAD, DH), jnp.float32),
            pltpu.VMEM_SHARED((N_PAD, DEGW), jnp.float32),
            pltpu.SemaphoreType.DMA,
            [pltpu.SemaphoreType.DMA] * NBUF,
            [pltpu.SemaphoreType.DMA] * NBUF,
            pltpu.SemaphoreType.DMA,
        ],
    )
    def agg_kernel(h_hbm, src_hbm, dst_hbm, zagg_hbm, zdeg_hbm,
                   agg_out, deg_out,
                   src_v, dst_v, rows_v, ones_v,
                   h_sh, agg_sh, deg_sh, isem, gsems, ssems, dsem):
        cid = lax.axis_index("c")
        sid = lax.axis_index("s")
        deg_half = groups_per_tile // 2

        ones16 = jnp.ones((16,), jnp.float32)

        def fill_ones(i, _):
            ones_v[i] = ones16
            return 0

        lax.fori_loop(0, CHUNK, fill_ones, 0)

        pltpu.async_copy(src_hbm.at[sid, pl.ds(0, GRP)], src_v.at[0], isem)
        pltpu.async_copy(dst_hbm.at[sid, pl.ds(0, GRP)], dst_v.at[0], isem)

        base = sid * BAND
        col = cid * DH

        @pl.when(base + BAND <= N_NODES)
        def _stage_full():
            pltpu.async_copy(h_hbm.at[pl.ds(base, BAND), pl.ds(col, DH)],
                             h_sh.at[pl.ds(base, BAND)], gsems[0]).wait()

        @pl.when(base + BAND > N_NODES)
        def _stage_tail():
            tail = N_NODES - (NS - 1) * BAND
            pltpu.async_copy(
                h_hbm.at[pl.ds((NS - 1) * BAND, tail), pl.ds(col, DH)],
                h_sh.at[pl.ds((NS - 1) * BAND, tail)], gsems[0]).wait()
        za = pltpu.async_copy(zagg_hbm, agg_sh.at[pl.ds(base, BAND)], gsems[1])
        zd = pltpu.async_copy(zdeg_hbm, deg_sh.at[pl.ds(base, BAND)], gsems[2])
        za.wait()
        zd.wait()
        plsc.subcore_barrier()

        def group(g, _):
            cur = lax.rem(g, 2)
            sv = src_v.at[cur]
            dv = dst_v.at[cur]
            pltpu.make_async_copy(
                src_hbm.at[sid, pl.ds(g * GRP, GRP)], sv, isem).wait()
            pltpu.make_async_copy(
                dst_hbm.at[sid, pl.ds(g * GRP, GRP)], dv, isem).wait()

            pv = dst_v.at[1 - cur]

            @pl.when(g > 0)
            def _wait_prev_scatters():
                for b in range(NBUF):
                    pltpu.make_async_copy(
                        rows_v.at[b], agg_sh.at[pv.at[GRP - NBUF + b]],
                        ssems[b]).wait()

            @pl.when(g + 1 < groups_per_tile)
            def _prefetch():
                nxt = 1 - cur
                pltpu.async_copy(
                    src_hbm.at[sid, pl.ds((g + 1) * GRP, GRP)],
                    src_v.at[nxt], isem)
                pltpu.async_copy(
                    dst_hbm.at[sid, pl.ds((g + 1) * GRP, GRP)],
                    dst_v.at[nxt], isem)

            do_deg = lax.select(cid == 0, g < deg_half, g >= deg_half)

            @pl.when(do_deg)
            def _deg_scatter():
                for j in range(GRP):
                    pltpu.async_copy(ones_v, deg_sh.at[dv.at[j]], dsem,
                                     add=True)

            def gather(j, b):
                return pltpu.async_copy(
                    h_sh.at[sv.at[j]], rows_v.at[b], gsems[b])

            gd = {}
            sd = {}
            for j in range(NBUF - 1):
                gd[j] = gather(j, j)
            for j in range(GRP):
                b = j % NBUF
                jn = j + NBUF - 1
                if jn < GRP:
                    bn = jn % NBUF
                    if j >= 1:
                        sd[j - 1].wait()
                    gd[jn] = gather(jn, bn)
                gd[j].wait()
                sd[j] = pltpu.async_copy(
                    rows_v.at[b], agg_sh.at[dv.at[j]], ssems[b], add=True)

            @pl.when(do_deg)
            def _deg_drain():
                for j in range(GRP):
                    pltpu.make_async_copy(
                        ones_v, deg_sh.at[dv.at[j]], dsem).wait()

            return 0

        lax.fori_loop(0, groups_per_tile, group, 0)

        lv = dst_v.at[lax.rem(groups_per_tile - 1, 2)]
        for b in range(NBUF):
            pltpu.make_async_copy(
                rows_v.at[b], agg_sh.at[lv.at[GRP - NBUF + b]],
                ssems[b]).wait()
        plsc.subcore_barrier()

        pltpu.sync_copy(agg_sh.at[pl.ds(base, BAND)],
                        agg_out.at[cid, pl.ds(base, BAND)])
        pltpu.sync_copy(deg_sh.at[pl.ds(base, BAND)],
                        deg_out.at[cid, pl.ds(base, BAND)])

    zagg = jnp.zeros((BAND, DH), jnp.float32)
    zdeg = jnp.zeros((BAND, DEGW), jnp.float32)
    return agg_kernel(hsplit, src, dst, zagg, zdeg)


def _tc_body(h_ref, a0_ref, a1_ref, d0_ref, d1_ref, w_ref, b_ref, o_ref):
    deg = d0_ref[0, :, 0:1] + d1_ref[0, :, 0:1]
    scale = 1.0 / jnp.maximum(deg, 1.0)
    gate = deg > 0.0
    hn0 = jnp.where(gate, a0_ref[0] * scale, h_ref[:, 0:DH])
    hn1 = jnp.where(gate, a1_ref[0] * scale, h_ref[:, DH:D])
    acc = (jnp.dot(hn0, w_ref[0:DH, :], preferred_element_type=jnp.float32)
           + jnp.dot(hn1, w_ref[DH:D, :], preferred_element_type=jnp.float32))
    o_ref[...] = jnp.maximum(acc + b_ref[...], 0.0)


def _tc_update(h, agg, deg, W, b):
    R = 2000
    grid = (N_NODES // R,)
    return pl.pallas_call(
        _tc_body,
        grid=grid,
        in_specs=[
            pl.BlockSpec((R, D), lambda i: (i, 0)),
            pl.BlockSpec((1, R, DH), lambda i: (0, i, 0)),
            pl.BlockSpec((1, R, DH), lambda i: (1, i, 0)),
            pl.BlockSpec((1, R, DEGW), lambda i: (0, i, 0)),
            pl.BlockSpec((1, R, DEGW), lambda i: (1, i, 0)),
            pl.BlockSpec((D, D), lambda i: (0, 0)),
            pl.BlockSpec((1, D), lambda i: (0, 0)),
        ],
        out_specs=pl.BlockSpec((R, D), lambda i: (i, 0)),
        out_shape=jax.ShapeDtypeStruct((N_NODES, D), jnp.float32),
    )(h, agg, agg, deg, deg, W, b)


def kernel(h, edge_index, W, b):
    src = edge_index[0].astype(jnp.int32)
    dst = edge_index[1].astype(jnp.int32)
    E = src.shape[0]
    lane = NS * GRP * CHUNK
    groups_per_tile = -(-E // lane)
    e_pad = lane * groups_per_tile
    if e_pad != E:
        src = jnp.concatenate(
            [src, jnp.zeros((e_pad - E,), jnp.int32)])
        dst = jnp.concatenate(
            [dst, jnp.full((e_pad - E,), N_NODES, jnp.int32)])
    src = src.reshape(NS, groups_per_tile * GRP, CHUNK)
    dst = dst.reshape(NS, groups_per_tile * GRP, CHUNK)

    agg_p, deg_p = _sc_aggregate(h, src, dst, groups_per_tile)

    return _tc_update(h, agg_p, deg_p, W, b.reshape(1, D))
